# baseline (speedup 1.0000x reference)
import functools

import jax
import jax.numpy as jnp
from jax import lax
from jax.experimental import pallas as pl
from jax.experimental.pallas import tpu as pltpu
from jax.experimental.pallas import tpu_sc as plsc

D, H, E, K = 768, 1024, 8, 2
T = 4096
P = T * K
BLK = 128
TILE = 512
NT = P // TILE + (E - 1)
XS = NT * TILE
NC, NS = 2, 16
NW = NC * NS
LANES = 16
WCOL = 128


_RC_Q, _RC_L = 64, 128
_CE = _RC_L * E


def _route_body(xf_ref, Wr_ref, br_ref, rank_ref, wts_ref, te_ref, meta_ref):
    xf = xf_ref[...]
    logits = lax.dot_general(Wr_ref[...], xf, (((0,), (1,)), ((), ())),
                             preferred_element_type=jnp.float32)
    logits = logits + br_ref[...]
    probs = jax.nn.softmax(logits, axis=0)
    rows = lax.broadcasted_iota(jnp.int32, (E, T), 0)
    p1 = jnp.max(probs, axis=0, keepdims=True)
    a1 = jnp.min(jnp.where(probs == p1, rows, E), axis=0, keepdims=True)
    masked = jnp.where(rows == a1, -jnp.inf, probs)
    p2 = jnp.max(masked, axis=0, keepdims=True)
    a2 = jnp.min(jnp.where(masked == p2, rows, E), axis=0, keepdims=True)
    denom = p1 + p2
    w_all = jnp.concatenate([(p1 / denom)[0], (p2 / denom)[0]], 0)
    wts_ref[...] = jnp.broadcast_to(w_all[:, None], (P, WCOL))

    e_all = jnp.concatenate([a1[0], a2[0]], 0)
    em = jnp.concatenate(
        [e_all[q * _RC_L:(q + 1) * _RC_L][None, :] for q in range(_RC_Q)], 0
    ).astype(jnp.float32)
    cl = lax.broadcasted_iota(jnp.int32, (_RC_L, _CE), 0)
    gl = lax.broadcasted_iota(jnp.int32, (_RC_L, _CE), 1) // E
    rep = (cl == gl).astype(jnp.float32)
    emx = jnp.dot(em, rep, preferred_element_type=jnp.float32)
    lane_e = lax.broadcasted_iota(jnp.int32, (_RC_Q, _CE), 1) % E
    oh = (emx == lane_e.astype(jnp.float32)).astype(jnp.float32)

    rq = lax.broadcasted_iota(jnp.int32, (_RC_Q, _RC_Q), 0)
    cq = lax.broadcasted_iota(jnp.int32, (_RC_Q, _RC_Q), 1)
    tri64 = (rq > cq).astype(jnp.float32)
    within = jnp.dot(tri64, oh, preferred_element_type=jnp.float32)

    colsums = jnp.sum(oh, axis=0, keepdims=True)
    ma = lax.broadcasted_iota(jnp.int32, (_CE, _CE), 0)
    mb = lax.broadcasted_iota(jnp.int32, (_CE, _CE), 1)
    m_sel = ((ma % E == mb % E) & (ma // E < mb // E)).astype(jnp.float32)
    ex = jnp.dot(colsums, m_sel, preferred_element_type=jnp.float32)

    bt = (lax.broadcasted_iota(jnp.int32, (_CE, E), 0) % E
          == lax.broadcasted_iota(jnp.int32, (_CE, E), 1)).astype(jnp.float32)
    total = jnp.dot(colsums, bt, preferred_element_type=jnp.float32)[0]
    pc = (((total.astype(jnp.int32) + TILE - 1) // TILE) * TILE).astype(jnp.float32)
    r8 = lax.broadcasted_iota(jnp.int32, (E, E), 0)
    c8 = lax.broadcasted_iota(jnp.int32, (E, E), 1)
    tri8 = (r8 >= c8).astype(jnp.float32)
    cumpc = jnp.dot(tri8, pc[:, None], preferred_element_type=jnp.float32)
    poff = (cumpc - pc[:, None])[:, 0]

    ti = lax.broadcasted_iota(jnp.int32, (BLK, E), 0) * TILE
    cumpc_i = cumpc[:, 0].astype(jnp.int32)
    te = jnp.sum((ti >= cumpc_i[None, :]).astype(jnp.int32), axis=1)
    tec = jnp.minimum(te, E - 1)
    te_ref[0, :] = tec

    used = cumpc_i[E - 1] // TILE
    ii = lax.broadcasted_iota(jnp.int32, (1, BLK), 1)[0]
    te_prev = jnp.concatenate([tec[:1], tec[:-1]], 0)
    bnd = (((ii == 0) | (tec != te_prev)) & (ii < used)).astype(jnp.float32)
    lo = lax.broadcasted_iota(jnp.int32, (BLK, BLK), 0)
    hi = lax.broadcasted_iota(jnp.int32, (BLK, BLK), 1)
    triu = (lo <= hi).astype(jnp.float32)
    seg = lax.dot_general(bnd[None, :], triu, (((1,), (0,)), ((), ())),
                          precision=lax.Precision.HIGHEST,
                          preferred_element_type=jnp.float32)[0].astype(jnp.int32)
    nseg = jnp.sum(bnd).astype(jnp.int32)
    par = (seg - 1) % 2
    isu = ((bnd > 0) & (seg < nseg)).astype(jnp.int32)
    isu2 = jnp.broadcast_to((nseg >= 2).astype(jnp.int32)[None], (BLK,))
    er = lax.broadcasted_iota(jnp.int32, (E, E), 0)
    ec = lax.broadcasted_iota(jnp.int32, (E, E), 1)
    pc_pos = (pc > 0)[None, :]
    ne = jnp.min(jnp.where((ec > er) & pc_pos, ec, E), axis=1)
    ne = jnp.where(ne == E, 0, ne).astype(jnp.float32)
    oh_te = (tec[:, None] == lax.broadcasted_iota(jnp.int32, (BLK, E), 1)
             ).astype(jnp.float32)
    nxt = jnp.dot(oh_te, ne[:, None],
                  preferred_element_type=jnp.float32)[:, 0].astype(jnp.int32)
    meta_ref[0, :] = bnd.astype(jnp.int32)
    meta_ref[1, :] = par
    meta_ref[2, :] = nxt
    meta_ref[3, :] = isu
    meta_ref[4, :] = isu2
    meta_ref[5, :] = jnp.broadcast_to(used[None], (BLK,))

    b_exp = (lax.broadcasted_iota(jnp.int32, (E, _CE), 0)
             == lax.broadcasted_iota(jnp.int32, (E, _CE), 1) % E).astype(jnp.float32)
    poff_row = jnp.dot(poff[None, :], b_exp,
                       preferred_element_type=jnp.float32)
    contrib = oh * (poff_row + ex + within)
    s_rows = lax.broadcasted_iota(jnp.int32, (_CE, _RC_L), 0) // E
    s_cols = lax.broadcasted_iota(jnp.int32, (_CE, _RC_L), 1)
    sel = (s_rows == s_cols).astype(jnp.float32)
    rank = lax.dot_general(contrib, sel, (((1,), (0,)), ((), ())),
                           precision=lax.Precision.HIGHEST,
                           preferred_element_type=jnp.float32)
    rank_ref[...] = rank.astype(jnp.int32)


def _route(xf, Wr, br2):
    return pl.pallas_call(
        _route_body,
        out_shape=[
            jax.ShapeDtypeStruct((_RC_Q, _RC_L), jnp.int32),
            jax.ShapeDtypeStruct((P, WCOL), jnp.float32),
            jax.ShapeDtypeStruct((1, BLK), jnp.int32),
            jax.ShapeDtypeStruct((6, BLK), jnp.int32),
        ],
    )(xf, Wr, br2)


_PAIRS_PER_W = P // NW
_DCH = 64
_NDCH = _PAIRS_PER_W // _DCH


def _make_dispatch(mesh):
  return functools.partial(
    pl.kernel,
    out_type=[
        jax.ShapeDtypeStruct((XS, D), jnp.float32),
        jax.ShapeDtypeStruct((XS, WCOL), jnp.float32),
    ],
    mesh=mesh,
    scratch_types=[
        pltpu.VMEM((_DCH,), jnp.int32),
        pltpu.VMEM((_DCH,), jnp.int32),
        pltpu.VMEM((_DCH, D), jnp.float32),
        pltpu.VMEM((_DCH, D), jnp.float32),
        pltpu.VMEM((_DCH, WCOL), jnp.float32),
        pltpu.VMEM((_DCH, WCOL), jnp.float32),
        pltpu.SemaphoreType.DMA,
        pltpu.SemaphoreType.DMA,
    ],
)(_dispatch_body)


def _dispatch_body(xf_hbm, rank_hbm, wts_hbm, xs_hbm, ws_hbm,
              i0_v, i1_v, r0_v, r1_v, w0_v, w1_v, sem0, sem1):
    wid = lax.axis_index("s") * NC + lax.axis_index("c")
    idx_b = (i0_v, i1_v)
    rows_b = (r0_v, r1_v)
    wv_b = (w0_v, w1_v)
    sem_b = (sem0, sem1)
    for ch in range(_NDCH):
        b = ch % 2
        pbase = wid * _PAIRS_PER_W + ch * _DCH
        tbase = lax.rem(pbase, T)
        if ch >= 2:
            pltpu.make_async_copy(rows_b[b], xs_hbm.at[idx_b[b]], sem_b[b]).wait()
            pltpu.make_async_copy(wv_b[b], ws_hbm.at[idx_b[b]], sem_b[b]).wait()
        pltpu.sync_copy(rank_hbm.at[pl.ds(pbase, _DCH)], idx_b[b])
        pltpu.sync_copy(xf_hbm.at[pl.ds(tbase, _DCH)], rows_b[b])
        pltpu.sync_copy(wts_hbm.at[pl.ds(pbase, _DCH)], wv_b[b])
        pltpu.async_copy(rows_b[b], xs_hbm.at[idx_b[b]], sem_b[b])
        pltpu.async_copy(wv_b[b], ws_hbm.at[idx_b[b]], sem_b[b])
    for ch in (_NDCH - 2, _NDCH - 1):
        b = ch % 2
        pltpu.make_async_copy(rows_b[b], xs_hbm.at[idx_b[b]], sem_b[b]).wait()
        pltpu.make_async_copy(wv_b[b], ws_hbm.at[idx_b[b]], sem_b[b]).wait()


def _glu_body(te_ref, fi_ref, pa_ref, nx_ref, isu_ref, isu2_ref, us_ref,
              xs_ref, ws_ref, W1_ref, W3_ref, W2_ref, out_ref,
              w1b, w3b, w2b, sem0, sem1):
    i = pl.program_id(0)
    e = te_ref[i]
    first = fi_ref[i]
    par = pa_ref[i]
    nxe = nx_ref[i]

    def fetch(eidx, slot, sem):
        pltpu.make_async_copy(W1_ref.at[eidx], w1b.at[slot], sem).start()
        pltpu.make_async_copy(W3_ref.at[eidx], w3b.at[slot], sem).start()
        pltpu.make_async_copy(W2_ref.at[eidx], w2b.at[slot], sem).start()

    def drain(eidx, slot, sem):
        pltpu.make_async_copy(W1_ref.at[eidx], w1b.at[slot], sem).wait()
        pltpu.make_async_copy(W3_ref.at[eidx], w3b.at[slot], sem).wait()
        pltpu.make_async_copy(W2_ref.at[eidx], w2b.at[slot], sem).wait()

    @pl.when(i == 0)
    def _():
        fetch(e, 0, sem0)

        @pl.when(isu2_ref[0] == 1)
        def _():
            fetch(nxe, 1, sem1)

    @pl.when((i > 0) & (first == 1) & (isu_ref[i] == 1))
    def _():
        @pl.when(par == 0)
        def _():
            fetch(nxe, 1, sem1)

        @pl.when(par == 1)
        def _():
            fetch(nxe, 0, sem0)

    @pl.when((first == 1) & (par == 0))
    def _():
        drain(e, 0, sem0)

    @pl.when((first == 1) & (par == 1))
    def _():
        drain(e, 1, sem1)

    @pl.when(i < us_ref[0])
    def _():
        xs = xs_ref[...]
        h1 = jnp.dot(xs, w1b[par], preferred_element_type=jnp.float32)
        h3 = jnp.dot(xs, w3b[par], preferred_element_type=jnp.float32)
        y = jnp.dot(jax.nn.silu(h1) * h3, w2b[par],
                    preferred_element_type=jnp.float32)
        out_ref[...] = y * ws_ref[...][:, :1]


def _glu(te, fi, pa, nx, isu, isu2, us, xs, ws, W1, W3, W2):
    grid_spec = pltpu.PrefetchScalarGridSpec(
        num_scalar_prefetch=7,
        grid=(NT,),
        in_specs=[
            pl.BlockSpec((TILE, D), lambda i, *_: (i, 0)),
            pl.BlockSpec((TILE, WCOL), lambda i, *_: (i, 0)),
            pl.BlockSpec(memory_space=pl.ANY),
            pl.BlockSpec(memory_space=pl.ANY),
            pl.BlockSpec(memory_space=pl.ANY),
        ],
        out_specs=pl.BlockSpec((TILE, D), lambda i, *_: (i, 0)),
        scratch_shapes=[
            pltpu.VMEM((2, D, H), jnp.float32),
            pltpu.VMEM((2, D, H), jnp.float32),
            pltpu.VMEM((2, H, D), jnp.float32),
            pltpu.SemaphoreType.DMA,
            pltpu.SemaphoreType.DMA,
        ],
    )
    return pl.pallas_call(
        _glu_body,
        grid_spec=grid_spec,
        out_shape=jax.ShapeDtypeStruct((XS, D), jnp.float32),
        compiler_params=pltpu.CompilerParams(
            dimension_semantics=("arbitrary",)),
    )(te, fi, pa, nx, isu, isu2, us, xs, ws, W1, W3, W2)


_TOK_PER_W = T // NW
_CCH = 64


def _make_combine(mesh):
  return functools.partial(
    pl.kernel,
    out_type=jax.ShapeDtypeStruct((T, D), jnp.float32),
    mesh=mesh,
    scratch_types=[
        pltpu.VMEM((_CCH,), jnp.int32),
        pltpu.VMEM((_CCH,), jnp.int32),
        pltpu.VMEM((_CCH, D), jnp.float32),
        pltpu.VMEM((_CCH, D), jnp.float32),
        pltpu.SemaphoreType.DMA,
    ],
)(_combine_body)


def _combine_body(os_hbm, rank_hbm, out_hbm, i0_v, i1_v, ra_v, rb_v, sem):
    wid = lax.axis_index("s") * NC + lax.axis_index("c")
    for ch in range(_TOK_PER_W // _CCH):
        tb = wid * _TOK_PER_W + ch * _CCH
        pltpu.sync_copy(rank_hbm.at[pl.ds(tb, _CCH)], i0_v)
        pltpu.sync_copy(rank_hbm.at[pl.ds(T + tb, _CCH)], i1_v)
        pltpu.async_copy(os_hbm.at[i0_v], ra_v, sem)
        pltpu.async_copy(os_hbm.at[i1_v], rb_v, sem)
        pltpu.make_async_copy(os_hbm.at[i0_v], ra_v, sem).wait()
        pltpu.make_async_copy(os_hbm.at[i1_v], rb_v, sem).wait()

        @plsc.parallel_loop(0, _CCH, unroll=2)
        def _(j):
            for i in range(D // LANES):
                sl = pl.ds(i * LANES, LANES)
                ra_v[j, sl] = ra_v[j, sl] + rb_v[j, sl]
        pltpu.sync_copy(ra_v, out_hbm.at[pl.ds(tb, _CCH)])


@functools.lru_cache(maxsize=None)
def _sc_kernels():
    mesh = plsc.VectorSubcoreMesh(
        core_axis_name="c", subcore_axis_name="s",
        num_cores=NC, num_subcores=NS)
    return _make_dispatch(mesh), _make_combine(mesh)


@jax.jit
def _moe(xf, Wr, br2, W1, W3, W2):
    rank2d, wts, te2d, meta = _route(xf, Wr, br2)
    rank = rank2d.reshape(P)
    te = te2d.reshape(BLK)
    fi, pa, nx, isu, isu2, us = (meta[j] for j in range(6))
    dispatch_k, combine_k = _sc_kernels()
    xs, ws = dispatch_k(xf, rank, wts)
    os_ = _glu(te, fi, pa, nx, isu, isu2, us, xs, ws, W1, W3, W2)
    return combine_k(os_, rank)


def kernel(x, Wr, br, W1, W3, W2):
    b, s, d = x.shape
    xf = x.reshape(b * s, d)
    out = _moe(xf, Wr, br.reshape(E, 1), W1, W3, W2)
    return out.reshape(b, s, d)


# speedup vs baseline: 1.0319x; 1.0319x over previous
import functools

import jax
import jax.numpy as jnp
from jax import lax
from jax.experimental import pallas as pl
from jax.experimental.pallas import tpu as pltpu
from jax.experimental.pallas import tpu_sc as plsc

D, H, E, K = 768, 1024, 8, 2
T = 4096
P = T * K
BLK = 128
TILE = 512
NT = P // TILE + (E - 1)
XS = NT * TILE
NC, NS = 2, 16
NW = NC * NS
LANES = 16
WCOL = 128


_RC_Q, _RC_L = 64, 128
_CE = _RC_L * E


def _route_body(xf_ref, Wr_ref, br_ref, rank_ref, wts_ref, te_ref, meta_ref):
    xf = xf_ref[...]
    logits = lax.dot_general(Wr_ref[...], xf, (((0,), (1,)), ((), ())),
                             preferred_element_type=jnp.float32)
    logits = logits + br_ref[...]
    probs = jax.nn.softmax(logits, axis=0)
    rows = lax.broadcasted_iota(jnp.int32, (E, T), 0)
    p1 = jnp.max(probs, axis=0, keepdims=True)
    a1 = jnp.min(jnp.where(probs == p1, rows, E), axis=0, keepdims=True)
    masked = jnp.where(rows == a1, -jnp.inf, probs)
    p2 = jnp.max(masked, axis=0, keepdims=True)
    a2 = jnp.min(jnp.where(masked == p2, rows, E), axis=0, keepdims=True)
    denom = p1 + p2
    w_all = jnp.concatenate([(p1 / denom)[0], (p2 / denom)[0]], 0)
    wts_ref[...] = jnp.broadcast_to(w_all[:, None], (P, WCOL))

    e_all = jnp.concatenate([a1[0], a2[0]], 0)
    em = jnp.concatenate(
        [e_all[q * _RC_L:(q + 1) * _RC_L][None, :] for q in range(_RC_Q)], 0
    ).astype(jnp.float32)
    cl = lax.broadcasted_iota(jnp.int32, (_RC_L, _CE), 0)
    gl = lax.broadcasted_iota(jnp.int32, (_RC_L, _CE), 1) // E
    rep = (cl == gl).astype(jnp.float32)
    emx = jnp.dot(em, rep, preferred_element_type=jnp.float32)
    lane_e = lax.broadcasted_iota(jnp.int32, (_RC_Q, _CE), 1) % E
    oh = (emx == lane_e.astype(jnp.float32)).astype(jnp.float32)

    rq = lax.broadcasted_iota(jnp.int32, (_RC_Q, _RC_Q), 0)
    cq = lax.broadcasted_iota(jnp.int32, (_RC_Q, _RC_Q), 1)
    tri64 = (rq > cq).astype(jnp.float32)
    within = jnp.dot(tri64, oh, preferred_element_type=jnp.float32)

    colsums = jnp.sum(oh, axis=0, keepdims=True)
    ma = lax.broadcasted_iota(jnp.int32, (_CE, _CE), 0)
    mb = lax.broadcasted_iota(jnp.int32, (_CE, _CE), 1)
    m_sel = ((ma % E == mb % E) & (ma // E < mb // E)).astype(jnp.float32)
    ex = jnp.dot(colsums, m_sel, preferred_element_type=jnp.float32)

    bt = (lax.broadcasted_iota(jnp.int32, (_CE, E), 0) % E
          == lax.broadcasted_iota(jnp.int32, (_CE, E), 1)).astype(jnp.float32)
    total = jnp.dot(colsums, bt, preferred_element_type=jnp.float32)[0]
    pc = (((total.astype(jnp.int32) + TILE - 1) // TILE) * TILE).astype(jnp.float32)
    r8 = lax.broadcasted_iota(jnp.int32, (E, E), 0)
    c8 = lax.broadcasted_iota(jnp.int32, (E, E), 1)
    tri8 = (r8 >= c8).astype(jnp.float32)
    cumpc = jnp.dot(tri8, pc[:, None], preferred_element_type=jnp.float32)
    poff = (cumpc - pc[:, None])[:, 0]

    ti = lax.broadcasted_iota(jnp.int32, (BLK, E), 0) * TILE
    cumpc_i = cumpc[:, 0].astype(jnp.int32)
    te = jnp.sum((ti >= cumpc_i[None, :]).astype(jnp.int32), axis=1)
    tec = jnp.minimum(te, E - 1)
    te_ref[0, :] = tec

    used = cumpc_i[E - 1] // TILE
    ii = lax.broadcasted_iota(jnp.int32, (1, BLK), 1)[0]
    te_prev = jnp.concatenate([tec[:1], tec[:-1]], 0)
    bnd = (((ii == 0) | (tec != te_prev)) & (ii < used)).astype(jnp.float32)
    lo = lax.broadcasted_iota(jnp.int32, (BLK, BLK), 0)
    hi = lax.broadcasted_iota(jnp.int32, (BLK, BLK), 1)
    triu = (lo <= hi).astype(jnp.float32)
    seg = lax.dot_general(bnd[None, :], triu, (((1,), (0,)), ((), ())),
                          precision=lax.Precision.HIGHEST,
                          preferred_element_type=jnp.float32)[0].astype(jnp.int32)
    nseg = jnp.sum(bnd).astype(jnp.int32)
    par = (seg - 1) % 2
    isu = ((bnd > 0) & (seg < nseg)).astype(jnp.int32)
    isu2 = jnp.broadcast_to((nseg >= 2).astype(jnp.int32)[None], (BLK,))
    er = lax.broadcasted_iota(jnp.int32, (E, E), 0)
    ec = lax.broadcasted_iota(jnp.int32, (E, E), 1)
    pc_pos = (pc > 0)[None, :]
    ne = jnp.min(jnp.where((ec > er) & pc_pos, ec, E), axis=1)
    ne = jnp.where(ne == E, 0, ne).astype(jnp.float32)
    oh_te = (tec[:, None] == lax.broadcasted_iota(jnp.int32, (BLK, E), 1)
             ).astype(jnp.float32)
    nxt = jnp.dot(oh_te, ne[:, None],
                  preferred_element_type=jnp.float32)[:, 0].astype(jnp.int32)
    meta_ref[0, :] = bnd.astype(jnp.int32)
    meta_ref[1, :] = par
    meta_ref[2, :] = nxt
    meta_ref[3, :] = isu
    meta_ref[4, :] = isu2
    meta_ref[5, :] = jnp.broadcast_to(used[None], (BLK,))

    b_exp = (lax.broadcasted_iota(jnp.int32, (E, _CE), 0)
             == lax.broadcasted_iota(jnp.int32, (E, _CE), 1) % E).astype(jnp.float32)
    poff_row = jnp.dot(poff[None, :], b_exp,
                       preferred_element_type=jnp.float32)
    contrib = oh * (poff_row + ex + within)
    s_rows = lax.broadcasted_iota(jnp.int32, (_CE, _RC_L), 0) // E
    s_cols = lax.broadcasted_iota(jnp.int32, (_CE, _RC_L), 1)
    sel = (s_rows == s_cols).astype(jnp.float32)
    rank = lax.dot_general(contrib, sel, (((1,), (0,)), ((), ())),
                           precision=lax.Precision.HIGHEST,
                           preferred_element_type=jnp.float32)
    rank_ref[...] = rank.astype(jnp.int32)


def _route(xf, Wr, br2):
    return pl.pallas_call(
        _route_body,
        out_shape=[
            jax.ShapeDtypeStruct((_RC_Q, _RC_L), jnp.int32),
            jax.ShapeDtypeStruct((P, WCOL), jnp.float32),
            jax.ShapeDtypeStruct((1, BLK), jnp.int32),
            jax.ShapeDtypeStruct((6, BLK), jnp.int32),
        ],
    )(xf, Wr, br2)


_PAIRS_PER_W = P // NW
_DCH = 64
_NDCH = _PAIRS_PER_W // _DCH


def _make_dispatch(mesh):
  return functools.partial(
    pl.kernel,
    out_type=[
        jax.ShapeDtypeStruct((XS, D), jnp.float32),
        jax.ShapeDtypeStruct((XS, WCOL), jnp.float32),
    ],
    mesh=mesh,
    scratch_types=[
        pltpu.VMEM((_DCH,), jnp.int32),
        pltpu.VMEM((_DCH,), jnp.int32),
        pltpu.VMEM((_DCH, D), jnp.float32),
        pltpu.VMEM((_DCH, D), jnp.float32),
        pltpu.VMEM((_DCH, WCOL), jnp.float32),
        pltpu.VMEM((_DCH, WCOL), jnp.float32),
        pltpu.SemaphoreType.DMA,
        pltpu.SemaphoreType.DMA,
    ],
)(_dispatch_body)


def _dispatch_body(xf_hbm, rank_hbm, wts_hbm, xs_hbm, ws_hbm,
              i0_v, i1_v, r0_v, r1_v, w0_v, w1_v, sem0, sem1):
    wid = lax.axis_index("s") * NC + lax.axis_index("c")
    idx_b = (i0_v, i1_v)
    rows_b = (r0_v, r1_v)
    wv_b = (w0_v, w1_v)
    sem_b = (sem0, sem1)
    for ch in range(_NDCH):
        b = ch % 2
        pbase = wid * _PAIRS_PER_W + ch * _DCH
        tbase = lax.rem(pbase, T)
        if ch >= 2:
            pltpu.make_async_copy(rows_b[b], xs_hbm.at[idx_b[b]], sem_b[b]).wait()
            pltpu.make_async_copy(wv_b[b], ws_hbm.at[idx_b[b]], sem_b[b]).wait()
        pltpu.sync_copy(rank_hbm.at[pl.ds(pbase, _DCH)], idx_b[b])
        pltpu.sync_copy(xf_hbm.at[pl.ds(tbase, _DCH)], rows_b[b])
        pltpu.sync_copy(wts_hbm.at[pl.ds(pbase, _DCH)], wv_b[b])
        pltpu.async_copy(rows_b[b], xs_hbm.at[idx_b[b]], sem_b[b])
        pltpu.async_copy(wv_b[b], ws_hbm.at[idx_b[b]], sem_b[b])
    for ch in (_NDCH - 2, _NDCH - 1):
        b = ch % 2
        pltpu.make_async_copy(rows_b[b], xs_hbm.at[idx_b[b]], sem_b[b]).wait()
        pltpu.make_async_copy(wv_b[b], ws_hbm.at[idx_b[b]], sem_b[b]).wait()


def _glu_body(te_ref, fi_ref, pa_ref, nx_ref, isu_ref, isu2_ref, us_ref,
              xs_ref, ws_ref, W1_ref, W3_ref, W2_ref, out_ref,
              w1b, w3b, w2b, sem0, sem1):
    i = pl.program_id(0)
    e = te_ref[i]
    first = fi_ref[i]
    par = pa_ref[i]
    nxe = nx_ref[i]

    def fetch(eidx, slot, sem):
        pltpu.make_async_copy(W1_ref.at[eidx], w1b.at[slot], sem).start()
        pltpu.make_async_copy(W3_ref.at[eidx], w3b.at[slot], sem).start()
        pltpu.make_async_copy(W2_ref.at[eidx], w2b.at[slot], sem).start()

    def drain(eidx, slot, sem):
        pltpu.make_async_copy(W1_ref.at[eidx], w1b.at[slot], sem).wait()
        pltpu.make_async_copy(W3_ref.at[eidx], w3b.at[slot], sem).wait()
        pltpu.make_async_copy(W2_ref.at[eidx], w2b.at[slot], sem).wait()

    @pl.when(i == 0)
    def _():
        fetch(e, 0, sem0)

        @pl.when(isu2_ref[0] == 1)
        def _():
            fetch(nxe, 1, sem1)

    @pl.when((i > 0) & (first == 1) & (isu_ref[i] == 1))
    def _():
        @pl.when(par == 0)
        def _():
            fetch(nxe, 1, sem1)

        @pl.when(par == 1)
        def _():
            fetch(nxe, 0, sem0)

    @pl.when((first == 1) & (par == 0))
    def _():
        drain(e, 0, sem0)

    @pl.when((first == 1) & (par == 1))
    def _():
        drain(e, 1, sem1)

    @pl.when(i < us_ref[0])
    def _():
        xs = xs_ref[...]
        h1 = jnp.dot(xs, w1b[par], preferred_element_type=jnp.float32)
        h3 = jnp.dot(xs, w3b[par], preferred_element_type=jnp.float32)
        y = jnp.dot(jax.nn.silu(h1) * h3, w2b[par],
                    preferred_element_type=jnp.float32)
        out_ref[...] = y * ws_ref[...][:, :1]


def _glu(te, fi, pa, nx, isu, isu2, us, xs, ws, W1, W3, W2):
    grid_spec = pltpu.PrefetchScalarGridSpec(
        num_scalar_prefetch=7,
        grid=(NT,),
        in_specs=[
            pl.BlockSpec((TILE, D), lambda i, *_: (i, 0)),
            pl.BlockSpec((TILE, WCOL), lambda i, *_: (i, 0)),
            pl.BlockSpec(memory_space=pl.ANY),
            pl.BlockSpec(memory_space=pl.ANY),
            pl.BlockSpec(memory_space=pl.ANY),
        ],
        out_specs=pl.BlockSpec((TILE, D), lambda i, *_: (i, 0)),
        scratch_shapes=[
            pltpu.VMEM((2, D, H), jnp.float32),
            pltpu.VMEM((2, D, H), jnp.float32),
            pltpu.VMEM((2, H, D), jnp.float32),
            pltpu.SemaphoreType.DMA,
            pltpu.SemaphoreType.DMA,
        ],
    )
    return pl.pallas_call(
        _glu_body,
        grid_spec=grid_spec,
        out_shape=jax.ShapeDtypeStruct((XS, D), jnp.float32),
        compiler_params=pltpu.CompilerParams(
            dimension_semantics=("arbitrary",)),
    )(te, fi, pa, nx, isu, isu2, us, xs, ws, W1, W3, W2)


_TOK_PER_W = T // NW
_CCH = 32
_NCCH = _TOK_PER_W // _CCH


def _make_combine(mesh):
  return functools.partial(
    pl.kernel,
    out_type=jax.ShapeDtypeStruct((T, D), jnp.float32),
    mesh=mesh,
    scratch_types=[
        pltpu.VMEM((_CCH,), jnp.int32),
        pltpu.VMEM((_CCH,), jnp.int32),
        pltpu.VMEM((_CCH,), jnp.int32),
        pltpu.VMEM((_CCH,), jnp.int32),
        pltpu.VMEM((_CCH, D), jnp.float32),
        pltpu.VMEM((_CCH, D), jnp.float32),
        pltpu.VMEM((_CCH, D), jnp.float32),
        pltpu.VMEM((_CCH, D), jnp.float32),
        pltpu.SemaphoreType.DMA,
        pltpu.SemaphoreType.DMA,
    ],
)(_combine_body)


def _combine_body(os_hbm, rank_hbm, out_hbm,
                  i0a, i1a, i0b, i1b, raa, rba, rab, rbb, sema, semb):
    wid = lax.axis_index("s") * NC + lax.axis_index("c")
    bufs = ((i0a, i1a, raa, rba, sema), (i0b, i1b, rab, rbb, semb))

    def issue(ch):
        i0, i1, ra, rb, sem = bufs[ch % 2]
        tb = wid * _TOK_PER_W + ch * _CCH
        pltpu.sync_copy(rank_hbm.at[pl.ds(tb, _CCH)], i0)
        pltpu.sync_copy(rank_hbm.at[pl.ds(T + tb, _CCH)], i1)
        pltpu.async_copy(os_hbm.at[i0], ra, sem)
        pltpu.async_copy(os_hbm.at[i1], rb, sem)

    issue(0)
    for ch in range(_NCCH):
        i0, i1, ra, rb, sem = bufs[ch % 2]
        tb = wid * _TOK_PER_W + ch * _CCH
        if ch + 1 < _NCCH:
            issue(ch + 1)
        pltpu.make_async_copy(os_hbm.at[i0], ra, sem).wait()
        pltpu.make_async_copy(os_hbm.at[i1], rb, sem).wait()

        def row_body(j, carry):
            for i in range(D // LANES):
                sl = pl.ds(i * LANES, LANES)
                ra[j, sl] = ra[j, sl] + rb[j, sl]
            return carry

        lax.fori_loop(0, _CCH, row_body, 0)
        pltpu.sync_copy(ra, out_hbm.at[pl.ds(tb, _CCH)])


@functools.lru_cache(maxsize=None)
def _sc_kernels():
    mesh = plsc.VectorSubcoreMesh(
        core_axis_name="c", subcore_axis_name="s",
        num_cores=NC, num_subcores=NS)
    return _make_dispatch(mesh), _make_combine(mesh)


@jax.jit
def _moe(xf, Wr, br2, W1, W3, W2):
    rank2d, wts, te2d, meta = _route(xf, Wr, br2)
    rank = rank2d.reshape(P)
    te = te2d.reshape(BLK)
    fi, pa, nx, isu, isu2, us = (meta[j] for j in range(6))
    dispatch_k, combine_k = _sc_kernels()
    xs, ws = dispatch_k(xf, rank, wts)
    os_ = _glu(te, fi, pa, nx, isu, isu2, us, xs, ws, W1, W3, W2)
    return combine_k(os_, rank)


def kernel(x, Wr, br, W1, W3, W2):
    b, s, d = x.shape
    xf = x.reshape(b * s, d)
    out = _moe(xf, Wr, br.reshape(E, 1), W1, W3, W2)
    return out.reshape(b, s, d)
